# baseline (speedup 1.0000x reference)
import jax
import jax.numpy as jnp
from jax.experimental import pallas as pl
from jax.experimental.pallas import tpu as pltpu


def _body(x_ref, r_ref, c_ref, o_ref):
    x = x_ref[0]
    r = r_ref[...]
    c = c_ref[...]
    half = r.shape[-1]
    o_ref[0, :, :, :half] = x[:, :, :half] + r[:, None, :]
    o_ref[0, :, :, half:] = x[:, :, half:] + c[None, :, :]


def kernel(inputs, row_emb, col_emb):
    B, R, Cg, C = inputs.shape
    out = pl.pallas_call(
        _body,
        grid=(B,),
        in_specs=[
            pl.BlockSpec((1, R, Cg, C), lambda b: (b, 0, 0, 0)),
            pl.BlockSpec((R, C // 2), lambda b: (0, 0)),
            pl.BlockSpec((Cg, C // 2), lambda b: (0, 0)),
        ],
        out_specs=pl.BlockSpec((1, R, Cg, C), lambda b: (b, 0, 0, 0)),
        out_shape=jax.ShapeDtypeStruct((B, R, Cg, C), inputs.dtype),
    )(inputs, row_emb, col_emb)
    return out.reshape(B, R * Cg, C)


# speedup vs baseline: 1.1928x; 1.1928x over previous
import jax
import jax.numpy as jnp
from jax.experimental import pallas as pl
from jax.experimental.pallas import tpu as pltpu


def _body(x_ref, r_ref, c_ref, o_ref):
    x = x_ref[...]
    r = r_ref[...]
    c = c_ref[...]
    half = r.shape[-1]
    o_ref[:, :, :, :half] = x[:, :, :, :half] + r[None, :, None, :]
    o_ref[:, :, :, half:] = x[:, :, :, half:] + c[None, None, :, :]


def kernel(inputs, row_emb, col_emb):
    B, R, Cg, C = inputs.shape
    BB = 4
    out = pl.pallas_call(
        _body,
        grid=(B // BB,),
        in_specs=[
            pl.BlockSpec((BB, R, Cg, C), lambda b: (b, 0, 0, 0)),
            pl.BlockSpec((R, C // 2), lambda b: (0, 0)),
            pl.BlockSpec((Cg, C // 2), lambda b: (0, 0)),
        ],
        out_specs=pl.BlockSpec((BB, R, Cg, C), lambda b: (b, 0, 0, 0)),
        out_shape=jax.ShapeDtypeStruct((B, R, Cg, C), inputs.dtype),
    )(inputs, row_emb, col_emb)
    return out.reshape(B, R * Cg, C)


# speedup vs baseline: 1.2275x; 1.0291x over previous
import jax
import jax.numpy as jnp
from jax.experimental import pallas as pl
from jax.experimental.pallas import tpu as pltpu


def _body(x_ref, r_ref, c_ref, o_ref):
    x = x_ref[...]
    r = r_ref[...]
    c = c_ref[...]
    half = r.shape[-1]
    o_ref[:, :, :, :half] = x[:, :, :, :half] + r[None, :, None, :]
    o_ref[:, :, :, half:] = x[:, :, :, half:] + c[None, None, :, :]


def kernel(inputs, row_emb, col_emb):
    B, R, Cg, C = inputs.shape
    BB = 8
    out = pl.pallas_call(
        _body,
        grid=(B // BB,),
        in_specs=[
            pl.BlockSpec((BB, R, Cg, C), lambda b: (b, 0, 0, 0)),
            pl.BlockSpec((R, C // 2), lambda b: (0, 0)),
            pl.BlockSpec((Cg, C // 2), lambda b: (0, 0)),
        ],
        out_specs=pl.BlockSpec((BB, R, Cg, C), lambda b: (b, 0, 0, 0)),
        out_shape=jax.ShapeDtypeStruct((B, R, Cg, C), inputs.dtype),
    )(inputs, row_emb, col_emb)
    return out.reshape(B, R * Cg, C)
